# baseline (speedup 1.0000x reference)
import functools

import jax
import jax.numpy as jnp
from jax import lax
from jax.experimental import pallas as pl
from jax.experimental.pallas import tpu as pltpu
from jax.experimental.pallas import tpu_sc as plsc

_DIM = 768
_E = 64
_K = 8
_N = 32768
_T = 256
_G = _N // _T
_FLAT = _N * _K
_NW = 32
_CHUNK = _FLAT // _NW
_LANES = 16


def _router_body(x_ref, wt_ref, w_out, e_out, r_out, cnt_out, base_out, run_ref):
    g = pl.program_id(0)

    @pl.when(g == 0)
    def _():
        run_ref[...] = jnp.zeros_like(run_ref)

    logits = jnp.dot(x_ref[...], wt_ref[...], preferred_element_type=jnp.float32)
    m = jnp.max(logits, axis=1, keepdims=True)
    p = jnp.exp(logits - m)
    denom = jnp.sum(p, axis=1, keepdims=True)

    lane = lax.broadcasted_iota(jnp.int32, (_T, _E), 1)
    lpri = (63 - lane).astype(jnp.float32)
    masks, vals = [], []
    sel = jnp.zeros((_T, _E), jnp.float32)
    cur = p
    for _ in range(_K):
        mx = jnp.max(cur, axis=1, keepdims=True)
        eqm = cur == mx
        tb = jnp.max(jnp.where(eqm, lpri, -1.0), axis=1, keepdims=True)
        mask = eqm & (lpri == tb)
        sel = sel + mask.astype(jnp.float32)
        cur = jnp.where(mask, -1.0, cur)
        masks.append(mask)
        vals.append(mx)

    rows = lax.broadcasted_iota(jnp.int32, (_T, _T), 0)
    cols = lax.broadcasted_iota(jnp.int32, (_T, _T), 1)
    tril = (rows > cols).astype(jnp.float32)
    prefix = jnp.dot(tril, sel, preferred_element_type=jnp.float32)
    rankmat = run_ref[...] + prefix

    payload = rankmat * 64.0 + lane.astype(jnp.float32)
    pays = [jnp.sum(jnp.where(mk, payload, 0.0), axis=1, keepdims=True)
            for mk in masks]
    pay8 = jnp.concatenate(pays, axis=1).astype(jnp.int32)

    w_out[...] = jnp.concatenate(vals, axis=1) / denom
    e_out[...] = pay8 & 63
    r_out[...] = pay8 >> 6

    counts_tile = jnp.sum(sel, axis=0, keepdims=True)
    new_run = run_ref[...] + counts_tile
    run_ref[...] = new_run

    @pl.when(g == _G - 1)
    def _():
        cnt = new_run.astype(jnp.int32)
        cnt_out[...] = cnt
        z1 = jnp.zeros((1, 1), jnp.int32)
        b = jnp.concatenate([z1, cnt[:, :-1]], axis=1)
        for sh in (1, 2, 4, 8, 16, 32):
            zs = jnp.zeros((1, sh), jnp.int32)
            b = b + jnp.concatenate([zs, b[:, :-sh]], axis=1)
        base_out[...] = b


_router_call = pl.pallas_call(
    _router_body,
    grid=(_G,),
    in_specs=[
        pl.BlockSpec((_T, _DIM), lambda g: (g, 0)),
        pl.BlockSpec((_DIM, _E), lambda g: (0, 0)),
    ],
    out_specs=[
        pl.BlockSpec((_T, _K), lambda g: (g, 0)),
        pl.BlockSpec((_T, _K), lambda g: (g, 0)),
        pl.BlockSpec((_T, _K), lambda g: (g, 0)),
        pl.BlockSpec((1, _E), lambda g: (0, 0)),
        pl.BlockSpec((1, _E), lambda g: (0, 0)),
    ],
    out_shape=[
        jax.ShapeDtypeStruct((_N, _K), jnp.float32),
        jax.ShapeDtypeStruct((_N, _K), jnp.int32),
        jax.ShapeDtypeStruct((_N, _K), jnp.int32),
        jax.ShapeDtypeStruct((1, _E), jnp.int32),
        jax.ShapeDtypeStruct((1, _E), jnp.int32),
    ],
    scratch_shapes=[pltpu.VMEM((1, _E), jnp.float32)],
    compiler_params=pltpu.CompilerParams(
        dimension_semantics=("arbitrary",)),
)


def _finalize_body(e_ref, r_ref, base_ref, s_out):
    lane = lax.broadcasted_iota(jnp.int32, (_T, _E), 1)
    base_row = base_ref[...]
    cols = []
    for k in range(_K):
        e_k = e_ref[:, k:k + 1]
        onehot = lane == e_k
        b_k = jnp.sum(jnp.where(onehot, base_row, 0), axis=1, keepdims=True)
        cols.append(r_ref[:, k:k + 1] + b_k)
    s_out[...] = jnp.concatenate(cols, axis=1)


_finalize_call = pl.pallas_call(
    _finalize_body,
    grid=(_G,),
    in_specs=[
        pl.BlockSpec((_T, _K), lambda g: (g, 0)),
        pl.BlockSpec((_T, _K), lambda g: (g, 0)),
        pl.BlockSpec((1, _E), lambda g: (0, 0)),
    ],
    out_specs=pl.BlockSpec((_T, _K), lambda g: (g, 0)),
    out_shape=jax.ShapeDtypeStruct((_N, _K), jnp.int32),
)


@functools.cache
def _permute_call():
    sub = _FLAT // 16
    wb = _FLAT // 32

    @functools.partial(
        pl.kernel,
        mesh=plsc.VectorSubcoreMesh(core_axis_name="c", subcore_axis_name="s",
                                    num_cores=2, num_subcores=16),
        out_type=jax.ShapeDtypeStruct((_FLAT,), jnp.int32),
        scratch_types=[
            pltpu.VMEM((sub,), jnp.int32),
            pltpu.VMEM((sub,), jnp.int32),
            pltpu.VMEM_SHARED((_FLAT,), jnp.int32),
        ],
    )
    def permute(s_hbm, iota_hbm, gather_hbm, sv, iv, buf):
        cid = lax.axis_index("c")
        sid = lax.axis_index("s")
        start = sid * sub
        pltpu.sync_copy(s_hbm.at[pl.ds(start, sub)], sv)
        pltpu.sync_copy(iota_hbm.at[pl.ds(start, sub)], iv)
        pltpu.sync_copy(iv, buf.at[sv])
        plsc.subcore_barrier()
        off = cid * (_FLAT // 2) + sid * wb
        pltpu.sync_copy(buf.at[pl.ds(off, wb)], gather_hbm.at[pl.ds(off, wb)])

    return permute


def kernel(x, W):
    x = x.reshape(-1, _DIM)
    w8, e8, r8, cnt, base = _router_call(x, W.T)
    scatter8 = _finalize_call(e8, r8, base)
    scatter = scatter8.reshape(-1)
    iota = jnp.arange(_FLAT, dtype=jnp.int32)
    gather = _permute_call()(scatter, iota)
    return w8.reshape(-1), gather, scatter, cnt.reshape(-1)


# speedup vs baseline: 1.4173x; 1.4173x over previous
import functools

import jax
import jax.numpy as jnp
from jax import lax
from jax.experimental import pallas as pl
from jax.experimental.pallas import tpu as pltpu
from jax.experimental.pallas import tpu_sc as plsc

_DIM = 768
_E = 64
_K = 8
_N = 32768
_T = 256
_G = _N // _T
_FLAT = _N * _K
_NW = 32
_CHUNK = _FLAT // _NW
_LANES = 16


def _router_body(x_ref, wt_ref, w_out, e_out, r_out, cnt_out, base_out, run_ref):
    g = pl.program_id(0)

    @pl.when(g == 0)
    def _():
        run_ref[...] = jnp.zeros_like(run_ref)

    logits = jnp.dot(x_ref[...], wt_ref[...], preferred_element_type=jnp.float32)
    m = jnp.max(logits, axis=1, keepdims=True)
    p = jnp.exp(logits - m)
    denom = jnp.sum(p, axis=1, keepdims=True)

    lane = lax.broadcasted_iota(jnp.int32, (_T, _E), 1)
    lpri = (63 - lane).astype(jnp.float32)
    masks, vals = [], []
    sel = jnp.zeros((_T, _E), jnp.float32)
    cur = p
    for _ in range(_K):
        mx = jnp.max(cur, axis=1, keepdims=True)
        eqm = cur == mx
        tb = jnp.max(jnp.where(eqm, lpri, -1.0), axis=1, keepdims=True)
        mask = eqm & (lpri == tb)
        sel = sel + mask.astype(jnp.float32)
        cur = jnp.where(mask, -1.0, cur)
        masks.append(mask)
        vals.append(mx)

    rows = lax.broadcasted_iota(jnp.int32, (_T, _T), 0)
    cols = lax.broadcasted_iota(jnp.int32, (_T, _T), 1)
    tril = (rows > cols).astype(jnp.float32)
    prefix = jnp.dot(tril, sel, preferred_element_type=jnp.float32)
    rankmat = run_ref[...] + prefix

    payload = rankmat * 64.0 + lane.astype(jnp.float32)
    pays = [jnp.sum(jnp.where(mk, payload, 0.0), axis=1, keepdims=True)
            for mk in masks]
    pay8 = jnp.concatenate(pays, axis=1).astype(jnp.int32)

    w_out[...] = jnp.concatenate(vals, axis=1) / denom
    e_out[...] = pay8 & 63
    r_out[...] = pay8 >> 6

    counts_tile = jnp.sum(sel, axis=0, keepdims=True)
    new_run = run_ref[...] + counts_tile
    run_ref[...] = new_run

    @pl.when(g == _G - 1)
    def _():
        cnt = new_run.astype(jnp.int32)
        cnt_out[...] = cnt
        z1 = jnp.zeros((1, 1), jnp.int32)
        b = jnp.concatenate([z1, cnt[:, :-1]], axis=1)
        for sh in (1, 2, 4, 8, 16, 32):
            zs = jnp.zeros((1, sh), jnp.int32)
            b = b + jnp.concatenate([zs, b[:, :-sh]], axis=1)
        base_out[...] = b


_router_call = pl.pallas_call(
    _router_body,
    grid=(_G,),
    in_specs=[
        pl.BlockSpec((_T, _DIM), lambda g: (g, 0)),
        pl.BlockSpec((_DIM, _E), lambda g: (0, 0)),
    ],
    out_specs=[
        pl.BlockSpec((_T, _K), lambda g: (g, 0)),
        pl.BlockSpec((_T, _K), lambda g: (g, 0)),
        pl.BlockSpec((_T, _K), lambda g: (g, 0)),
        pl.BlockSpec((1, _E), lambda g: (0, 0)),
        pl.BlockSpec((1, _E), lambda g: (0, 0)),
    ],
    out_shape=[
        jax.ShapeDtypeStruct((_N, _K), jnp.float32),
        jax.ShapeDtypeStruct((_N, _K), jnp.int32),
        jax.ShapeDtypeStruct((_N, _K), jnp.int32),
        jax.ShapeDtypeStruct((1, _E), jnp.int32),
        jax.ShapeDtypeStruct((1, _E), jnp.int32),
    ],
    scratch_shapes=[pltpu.VMEM((1, _E), jnp.float32)],
    compiler_params=pltpu.CompilerParams(
        dimension_semantics=("arbitrary",)),
)


@functools.cache
def _permute_call():
    sub = _FLAT // 16
    wb = _FLAT // 32

    @functools.partial(
        pl.kernel,
        mesh=plsc.VectorSubcoreMesh(core_axis_name="c", subcore_axis_name="s",
                                    num_cores=2, num_subcores=16),
        out_type=[
            jax.ShapeDtypeStruct((_FLAT,), jnp.int32),
            jax.ShapeDtypeStruct((_FLAT,), jnp.int32),
        ],
        scratch_types=[
            pltpu.VMEM((sub,), jnp.int32),
            pltpu.VMEM((sub,), jnp.int32),
            pltpu.VMEM((sub,), jnp.int32),
            pltpu.VMEM((sub,), jnp.int32),
            pltpu.VMEM((sub,), jnp.int32),
            pltpu.VMEM((_E,), jnp.int32),
            pltpu.VMEM_SHARED((_FLAT,), jnp.int32),
            pltpu.VMEM_SHARED((_E,), jnp.int32),
            pltpu.SemaphoreType.DMA,
        ],
    )
    def permute(e_hbm, r_hbm, b_hbm, iota_hbm, gather_hbm, scatter_hbm,
                ev, rv, bv, sv, iv, bt, buf, base_sh, sem):
        cid = lax.axis_index("c")
        sid = lax.axis_index("s")
        start = sid * sub

        @pl.when(sid == 0)
        def _():
            pltpu.sync_copy(b_hbm, bt)
            pltpu.sync_copy(bt, base_sh)

        pltpu.sync_copy(e_hbm.at[pl.ds(start, sub)], ev)
        pltpu.sync_copy(r_hbm.at[pl.ds(start, sub)], rv)
        pltpu.sync_copy(iota_hbm.at[pl.ds(start, sub)], iv)
        plsc.subcore_barrier()
        pltpu.async_copy(base_sh.at[ev], bv, sem).wait()

        def body(i, carry):
            off = i * _LANES
            sv[pl.ds(off, _LANES)] = bv[pl.ds(off, _LANES)] + rv[pl.ds(off, _LANES)]
            return carry

        lax.fori_loop(0, sub // _LANES, body, 0, unroll=8)

        @pl.when(cid == 0)
        def _():
            pltpu.sync_copy(sv, scatter_hbm.at[pl.ds(start, sub)])

        pltpu.sync_copy(iv, buf.at[sv])
        plsc.subcore_barrier()
        off = cid * (_FLAT // 2) + sid * wb
        pltpu.sync_copy(buf.at[pl.ds(off, wb)], gather_hbm.at[pl.ds(off, wb)])

    return permute


def kernel(x, W):
    x = x.reshape(-1, _DIM)
    w8, e8, r8, cnt, base = _router_call(x, W.T)
    iota = jnp.arange(_FLAT, dtype=jnp.int32)
    gather, scatter = _permute_call()(
        e8.reshape(-1), r8.reshape(-1), base.reshape(-1), iota)
    return w8.reshape(-1), gather, scatter, cnt.reshape(-1)


# speedup vs baseline: 1.4462x; 1.0204x over previous
import functools

import jax
import jax.numpy as jnp
from jax import lax
from jax.experimental import pallas as pl
from jax.experimental.pallas import tpu as pltpu
from jax.experimental.pallas import tpu_sc as plsc

_DIM = 768
_E = 64
_K = 8
_N = 32768
_T = 256
_G = _N // _T
_FLAT = _N * _K
_NW = 32
_CHUNK = _FLAT // _NW
_LANES = 16


def _router_body(x_ref, wt_ref, w_out, e_out, r_out, cnt_out, base_out, run_ref):
    g = pl.program_id(0)

    @pl.when(g == 0)
    def _():
        run_ref[...] = jnp.zeros_like(run_ref)

    logits = jnp.dot(x_ref[...], wt_ref[...], preferred_element_type=jnp.float32)
    m = jnp.max(logits, axis=1, keepdims=True)
    p = jnp.exp(logits - m)
    denom = jnp.sum(p, axis=1, keepdims=True)

    lane = lax.broadcasted_iota(jnp.int32, (_T, _E), 1)
    _H = _T // 2
    lpri_h = (63 - lane[:_H]).astype(jnp.float32)

    def _top8(pblk):
        cur = pblk
        masks, vals = [], []
        for _ in range(_K):
            mx = jnp.max(cur, axis=1, keepdims=True)
            eqm = cur == mx
            tb = jnp.max(jnp.where(eqm, lpri_h, -1.0), axis=1, keepdims=True)
            mask = lpri_h == tb
            cur = jnp.where(mask, -1.0, cur)
            masks.append(mask)
            vals.append(mx)
        return masks, vals, cur

    masks0, vals0, cur0 = _top8(p[:_H])
    masks1, vals1, cur1 = _top8(p[_H:])
    sel = jnp.concatenate([cur0 < 0.0, cur1 < 0.0], axis=0).astype(jnp.float32)

    rows = lax.broadcasted_iota(jnp.int32, (_T, _T), 0)
    cols = lax.broadcasted_iota(jnp.int32, (_T, _T), 1)
    tril = (rows > cols).astype(jnp.float32)
    prefix = jnp.dot(tril, sel, preferred_element_type=jnp.float32)
    rankmat = run_ref[...] + prefix

    payload = rankmat * 64.0 + lane.astype(jnp.float32)
    pays0 = [jnp.sum(jnp.where(mk, payload[:_H], 0.0), axis=1, keepdims=True)
             for mk in masks0]
    pays1 = [jnp.sum(jnp.where(mk, payload[_H:], 0.0), axis=1, keepdims=True)
             for mk in masks1]
    pay8 = jnp.concatenate([
        jnp.concatenate(pays0, axis=1),
        jnp.concatenate(pays1, axis=1)], axis=0).astype(jnp.int32)

    w_out[...] = jnp.concatenate([
        jnp.concatenate(vals0, axis=1),
        jnp.concatenate(vals1, axis=1)], axis=0) / denom
    e_out[...] = pay8 & 63
    r_out[...] = pay8 >> 6

    counts_tile = jnp.sum(sel, axis=0, keepdims=True)
    new_run = run_ref[...] + counts_tile
    run_ref[...] = new_run

    @pl.when(g == _G - 1)
    def _():
        cnt = new_run.astype(jnp.int32)
        cnt_out[...] = cnt
        z1 = jnp.zeros((1, 1), jnp.int32)
        b = jnp.concatenate([z1, cnt[:, :-1]], axis=1)
        for sh in (1, 2, 4, 8, 16, 32):
            zs = jnp.zeros((1, sh), jnp.int32)
            b = b + jnp.concatenate([zs, b[:, :-sh]], axis=1)
        base_out[...] = b


_router_call = pl.pallas_call(
    _router_body,
    grid=(_G,),
    in_specs=[
        pl.BlockSpec((_T, _DIM), lambda g: (g, 0)),
        pl.BlockSpec((_DIM, _E), lambda g: (0, 0)),
    ],
    out_specs=[
        pl.BlockSpec((_T, _K), lambda g: (g, 0)),
        pl.BlockSpec((_T, _K), lambda g: (g, 0)),
        pl.BlockSpec((_T, _K), lambda g: (g, 0)),
        pl.BlockSpec((1, _E), lambda g: (0, 0)),
        pl.BlockSpec((1, _E), lambda g: (0, 0)),
    ],
    out_shape=[
        jax.ShapeDtypeStruct((_N, _K), jnp.float32),
        jax.ShapeDtypeStruct((_N, _K), jnp.int32),
        jax.ShapeDtypeStruct((_N, _K), jnp.int32),
        jax.ShapeDtypeStruct((1, _E), jnp.int32),
        jax.ShapeDtypeStruct((1, _E), jnp.int32),
    ],
    scratch_shapes=[pltpu.VMEM((1, _E), jnp.float32)],
    compiler_params=pltpu.CompilerParams(
        dimension_semantics=("arbitrary",)),
)


@functools.cache
def _permute_call():
    sub = _FLAT // 16
    wb = _FLAT // 32

    @functools.partial(
        pl.kernel,
        mesh=plsc.VectorSubcoreMesh(core_axis_name="c", subcore_axis_name="s",
                                    num_cores=2, num_subcores=16),
        out_type=[
            jax.ShapeDtypeStruct((_FLAT,), jnp.int32),
            jax.ShapeDtypeStruct((_FLAT,), jnp.int32),
        ],
        scratch_types=[
            pltpu.VMEM((sub,), jnp.int32),
            pltpu.VMEM((sub,), jnp.int32),
            pltpu.VMEM((sub,), jnp.int32),
            pltpu.VMEM((sub,), jnp.int32),
            pltpu.VMEM((sub,), jnp.int32),
            pltpu.VMEM((_E,), jnp.int32),
            pltpu.VMEM_SHARED((_FLAT,), jnp.int32),
            pltpu.VMEM_SHARED((_E,), jnp.int32),
            pltpu.SemaphoreType.DMA,
        ],
    )
    def permute(e_hbm, r_hbm, b_hbm, iota_hbm, gather_hbm, scatter_hbm,
                ev, rv, bv, sv, iv, bt, buf, base_sh, sem):
        cid = lax.axis_index("c")
        sid = lax.axis_index("s")
        start = sid * sub

        @pl.when(sid == 0)
        def _():
            pltpu.sync_copy(b_hbm, bt)
            pltpu.sync_copy(bt, base_sh)

        pltpu.sync_copy(e_hbm.at[pl.ds(start, sub)], ev)
        pltpu.sync_copy(r_hbm.at[pl.ds(start, sub)], rv)
        pltpu.sync_copy(iota_hbm.at[pl.ds(start, sub)], iv)
        plsc.subcore_barrier()
        pltpu.async_copy(base_sh.at[ev], bv, sem).wait()

        def body(i, carry):
            off = i * _LANES
            sv[pl.ds(off, _LANES)] = bv[pl.ds(off, _LANES)] + rv[pl.ds(off, _LANES)]
            return carry

        lax.fori_loop(0, sub // _LANES, body, 0, unroll=8)

        @pl.when(cid == 0)
        def _():
            pltpu.sync_copy(sv, scatter_hbm.at[pl.ds(start, sub)])

        pltpu.sync_copy(iv, buf.at[sv])
        plsc.subcore_barrier()
        off = cid * (_FLAT // 2) + sid * wb
        pltpu.sync_copy(buf.at[pl.ds(off, wb)], gather_hbm.at[pl.ds(off, wb)])

    return permute


def kernel(x, W):
    x = x.reshape(-1, _DIM)
    w8, e8, r8, cnt, base = _router_call(x, W.T)
    iota = jnp.arange(_FLAT, dtype=jnp.int32)
    gather, scatter = _permute_call()(
        e8.reshape(-1), r8.reshape(-1), base.reshape(-1), iota)
    return w8.reshape(-1), gather, scatter, cnt.reshape(-1)


# speedup vs baseline: 2.2550x; 1.5592x over previous
import functools

import jax
import jax.numpy as jnp
from jax import lax
from jax.experimental import pallas as pl
from jax.experimental.pallas import tpu as pltpu
from jax.experimental.pallas import tpu_sc as plsc

_DIM = 768
_E = 64
_K = 8
_N = 32768
_T = 256
_G = _N // _T
_FLAT = _N * _K
_NW = 32
_CHUNK = _FLAT // _NW
_LANES = 16


def _router_body(x_ref, wt_ref, w_out, e_out, r_out, cnt_out, base_out, run_ref):
    g = pl.program_id(0)

    @pl.when(g == 0)
    def _():
        run_ref[...] = jnp.zeros_like(run_ref)

    logits = jnp.dot(x_ref[...], wt_ref[...], preferred_element_type=jnp.float32)
    lt = logits.T
    m = jnp.max(lt, axis=0, keepdims=True)
    p = jnp.exp(lt - m)
    denom = jnp.sum(p, axis=0, keepdims=True)

    laneT = lax.broadcasted_iota(jnp.int32, (_E, _T), 0)
    lpriT = (63 - laneT).astype(jnp.float32)
    masks, vals = [], []
    cur = p
    for _ in range(_K):
        mx = jnp.max(cur, axis=0, keepdims=True)
        eqm = cur == mx
        tb = jnp.max(jnp.where(eqm, lpriT, -1.0), axis=0, keepdims=True)
        mask = lpriT == tb
        cur = jnp.where(mask, -1.0, cur)
        masks.append(mask)
        vals.append(mx)
    selT = (cur < 0.0).astype(jnp.float32)

    rows = lax.broadcasted_iota(jnp.int32, (_T, _T), 0)
    cols = lax.broadcasted_iota(jnp.int32, (_T, _T), 1)
    sut = (rows < cols).astype(jnp.float32)
    prefixT = jnp.dot(selT, sut, preferred_element_type=jnp.float32)
    run_col = run_ref[...]
    rankmatT = run_col + prefixT

    payloadT = rankmatT * 64.0 + laneT.astype(jnp.float32)
    pays = [jnp.sum(jnp.where(mk, payloadT, 0.0), axis=0, keepdims=True)
            for mk in masks]
    pay8 = jnp.concatenate(pays, axis=0)
    e8T = lax.bitcast_convert_type(pay8.astype(jnp.int32) & 63, jnp.float32)
    r8T = lax.bitcast_convert_type(pay8.astype(jnp.int32) >> 6, jnp.float32)
    w8T = jnp.concatenate(vals, axis=0) / denom

    w_out[...] = w8T.T
    e_out[...] = lax.bitcast_convert_type(e8T.T, jnp.int32)
    r_out[...] = lax.bitcast_convert_type(r8T.T, jnp.int32)

    counts_tile = jnp.sum(selT, axis=1, keepdims=True)
    new_run = run_col + counts_tile
    run_ref[...] = new_run

    @pl.when(g == _G - 1)
    def _():
        cnt = new_run.astype(jnp.int32).T
        cnt_out[...] = cnt
        z1 = jnp.zeros((1, 1), jnp.int32)
        b = jnp.concatenate([z1, cnt[:, :-1]], axis=1)
        for sh in (1, 2, 4, 8, 16, 32):
            zs = jnp.zeros((1, sh), jnp.int32)
            b = b + jnp.concatenate([zs, b[:, :-sh]], axis=1)
        base_out[...] = b


_router_call = pl.pallas_call(
    _router_body,
    grid=(_G,),
    in_specs=[
        pl.BlockSpec((_T, _DIM), lambda g: (g, 0)),
        pl.BlockSpec((_DIM, _E), lambda g: (0, 0)),
    ],
    out_specs=[
        pl.BlockSpec((_T, _K), lambda g: (g, 0)),
        pl.BlockSpec((_T, _K), lambda g: (g, 0)),
        pl.BlockSpec((_T, _K), lambda g: (g, 0)),
        pl.BlockSpec((1, _E), lambda g: (0, 0)),
        pl.BlockSpec((1, _E), lambda g: (0, 0)),
    ],
    out_shape=[
        jax.ShapeDtypeStruct((_N, _K), jnp.float32),
        jax.ShapeDtypeStruct((_N, _K), jnp.int32),
        jax.ShapeDtypeStruct((_N, _K), jnp.int32),
        jax.ShapeDtypeStruct((1, _E), jnp.int32),
        jax.ShapeDtypeStruct((1, _E), jnp.int32),
    ],
    scratch_shapes=[pltpu.VMEM((_E, 1), jnp.float32)],
    compiler_params=pltpu.CompilerParams(
        dimension_semantics=("arbitrary",)),
)


@functools.cache
def _permute_call():
    sub = _FLAT // 16
    wb = _FLAT // 32

    @functools.partial(
        pl.kernel,
        mesh=plsc.VectorSubcoreMesh(core_axis_name="c", subcore_axis_name="s",
                                    num_cores=2, num_subcores=16),
        out_type=[
            jax.ShapeDtypeStruct((_FLAT,), jnp.int32),
            jax.ShapeDtypeStruct((_FLAT,), jnp.int32),
        ],
        scratch_types=[
            pltpu.VMEM((sub,), jnp.int32),
            pltpu.VMEM((sub,), jnp.int32),
            pltpu.VMEM((sub,), jnp.int32),
            pltpu.VMEM((sub,), jnp.int32),
            pltpu.VMEM((sub,), jnp.int32),
            pltpu.VMEM((_E,), jnp.int32),
            pltpu.VMEM_SHARED((_FLAT,), jnp.int32),
            pltpu.VMEM_SHARED((_E,), jnp.int32),
            pltpu.SemaphoreType.DMA,
        ],
    )
    def permute(e_hbm, r_hbm, b_hbm, iota_hbm, gather_hbm, scatter_hbm,
                ev, rv, bv, sv, iv, bt, buf, base_sh, sem):
        cid = lax.axis_index("c")
        sid = lax.axis_index("s")
        start = sid * sub

        @pl.when(sid == 0)
        def _():
            pltpu.sync_copy(b_hbm, bt)
            pltpu.sync_copy(bt, base_sh)

        pltpu.sync_copy(e_hbm.at[pl.ds(start, sub)], ev)
        pltpu.sync_copy(r_hbm.at[pl.ds(start, sub)], rv)
        pltpu.sync_copy(iota_hbm.at[pl.ds(start, sub)], iv)
        plsc.subcore_barrier()
        pltpu.async_copy(base_sh.at[ev], bv, sem).wait()

        def body(i, carry):
            off = i * _LANES
            sv[pl.ds(off, _LANES)] = bv[pl.ds(off, _LANES)] + rv[pl.ds(off, _LANES)]
            return carry

        lax.fori_loop(0, sub // _LANES, body, 0, unroll=8)

        @pl.when(cid == 0)
        def _():
            pltpu.sync_copy(sv, scatter_hbm.at[pl.ds(start, sub)])

        pltpu.sync_copy(iv, buf.at[sv])
        plsc.subcore_barrier()
        off = cid * (_FLAT // 2) + sid * wb
        pltpu.sync_copy(buf.at[pl.ds(off, wb)], gather_hbm.at[pl.ds(off, wb)])

    return permute


def kernel(x, W):
    x = x.reshape(-1, _DIM)
    w8, e8, r8, cnt, base = _router_call(x, W.T)
    iota = jnp.arange(_FLAT, dtype=jnp.int32)
    gather, scatter = _permute_call()(
        e8.reshape(-1), r8.reshape(-1), base.reshape(-1), iota)
    return w8.reshape(-1), gather, scatter, cnt.reshape(-1)
